# baseline (speedup 1.0000x reference)
import functools

import jax
import jax.numpy as jnp
from jax import lax
from jax.experimental import pallas as pl
from jax.experimental.pallas import tpu as pltpu
from jax.experimental.pallas import tpu_sc as plsc

_NC = 2
_NS = 16
_NW = _NC * _NS
_LANES = 16
_OPAD = 8


def _rne_bf16_bits(x):
    u = lax.bitcast_convert_type(x, jnp.int32)
    lsb = lax.shift_right_logical(u, 16) & 1
    return lax.shift_right_logical(u + 0x7FFF + lsb, 16)


def _make_proj(Vn, Dn, blk):
    grid = (Vn + blk - 1) // blk

    def body(t_ref, w_ref, o_ref):
        i = pl.program_id(0)
        cols = lax.broadcasted_iota(jnp.int32, (1, blk), 1) + i * blk
        t = jnp.where(cols == 0, 0.0, t_ref[...])
        o = lax.dot_general(w_ref[...], t, (((1,), (0,)), ((), ())),
                            preferred_element_type=jnp.float32)
        b0 = _rne_bf16_bits(o[0, :])
        b1 = _rne_bf16_bits(o[1, :])
        o_ref[...] = b0 | lax.shift_left(b1, 16)

    return pl.pallas_call(
        body,
        grid=(grid,),
        in_specs=[
            pl.BlockSpec((Dn, blk), lambda i: (0, i)),
            pl.BlockSpec((_OPAD, Dn), lambda i: (0, 0)),
        ],
        out_specs=pl.BlockSpec((blk,), lambda i: (i,)),
        out_shape=jax.ShapeDtypeStruct((Vn,), jnp.int32),
    )


def _make_pool(Bn, Ln):
    bpw = Bn // _NW
    mesh = plsc.VectorSubcoreMesh(
        core_axis_name="c", subcore_axis_name="s",
        num_cores=_NC, num_subcores=_NS)

    chunks = []
    off = 0
    while off < Ln:
        sz = min(128, Ln - off)
        chunks.append((off, sz))
        off += sz

    nacc = (Ln + _LANES - 1) // _LANES
    lpad = nacc * _LANES
    nbuf = 8
    assert bpw % nbuf == 0

    @functools.partial(
        pl.kernel,
        out_type=jax.ShapeDtypeStruct((Bn, _LANES), jnp.float32),
        mesh=mesh,
        scratch_types=[
            pltpu.VMEM((lpad, bpw), jnp.int32),
            pltpu.VMEM((bpw, lpad), jnp.int32),
            pltpu.VMEM((nbuf, lpad), jnp.int32),
            pltpu.VMEM((bpw,), jnp.int32),
            pltpu.VMEM((_LANES,), jnp.float32),
            pltpu.VMEM((bpw, _LANES), jnp.float32),
            [pltpu.SemaphoreType.DMA] * nbuf,
        ],
        compiler_params=pltpu.CompilerParams(
            use_tc_tiling_on_sc=False, needs_layout_passes=False),
    )
    def pool(data_t_hbm, len_hbm, bias_hbm, packed_hbm, out_hbm,
             idx_t, idx_v, rows_v, len_v, bias_v, out_v, sems):
        wid = lax.axis_index("s") * _NC + lax.axis_index("c")
        base = wid * bpw
        pltpu.sync_copy(data_t_hbm.at[:, pl.ds(base, bpw)],
                        idx_t.at[pl.ds(0, Ln)])
        pltpu.sync_copy(len_hbm.at[pl.ds(base, bpw)], len_v)
        pltpu.sync_copy(bias_hbm, bias_v)
        bias_vec = bias_v[...]
        lane = lax.iota(jnp.int32, _LANES)

        def transpose_body(g, carry):
            col = jnp.full((_LANES,), g, jnp.int32)
            for c in range(nacc):
                rowi = lax.iota(jnp.int32, _LANES) + c * _LANES
                idx_v[g, pl.ds(c * _LANES, _LANES)] = plsc.load_gather(
                    idx_t, [rowi, col])
            return carry

        lax.fori_loop(0, bpw, transpose_body, 0)

        izeros = jnp.zeros((_LANES,), jnp.int32)
        for slot in range(nbuf):
            for c in range(Ln // _LANES * _LANES, lpad, _LANES):
                rows_v[slot, pl.ds(c, _LANES)] = izeros

        def fire(g, slot, sem):
            gi = jnp.minimum(g, bpw - 1)
            for (o, sz) in chunks:
                pltpu.async_copy(
                    packed_hbm.at[idx_v.at[gi, pl.ds(o, sz)]],
                    rows_v.at[slot, pl.ds(o, sz)], sem)

        def drain(slot, sem):
            for (o, sz) in chunks:
                pltpu.make_async_copy(
                    packed_hbm.at[idx_v.at[0, pl.ds(o, sz)]],
                    rows_v.at[slot, pl.ds(o, sz)], sem).wait()

        fzeros = jnp.zeros((_LANES,), jnp.float32)
        himask = jnp.full((_LANES,), -65536, jnp.int32)

        def acc_store(g, slot):
            acc0 = fzeros
            acc1 = fzeros
            for c in range(nacc):
                w = rows_v[slot, pl.ds(c * _LANES, _LANES)]
                acc0 = acc0 + plsc.bitcast(lax.shift_left(w, 16), jnp.float32)
                acc1 = acc1 + plsc.bitcast(w & himask, jnp.float32)
            s0 = jnp.sum(acc0)
            s1 = jnp.sum(acc1)
            lenf = plsc.load_gather(
                len_v, [jnp.full((_LANES,), g, jnp.int32)]).astype(jnp.float32)
            res = jnp.where(lane == 0, s0, jnp.where(lane == 1, s1, 0.0))
            out_v[g, :] = res / lenf + bias_vec

        for j in range(nbuf - 1):
            fire(j, j, sems[j])

        def outer(i, carry):
            g = nbuf * i
            for j in range(nbuf):
                fire(g + nbuf - 1 + j, (j + nbuf - 1) % nbuf,
                     sems[(j + nbuf - 1) % nbuf])
                drain(j, sems[j])
                acc_store(g + j, j)
            return carry

        lax.fori_loop(0, bpw // nbuf, outer, 0)
        for j in range(nbuf - 1):
            drain(j, sems[j])

        pltpu.sync_copy(out_v, out_hbm.at[pl.ds(base, bpw)])

    return pool


def kernel(data, length, embed_table, W, b):
    Bn, Ln = data.shape
    Vn, Dn = embed_table.shape
    On = W.shape[0]
    data = data.astype(jnp.int32)

    Wp = jnp.zeros((_OPAD, Dn), jnp.float32).at[:On].set(W)
    packed = _make_proj(Vn, Dn, 65536)(embed_table.T, Wp)

    bias = jnp.zeros((_LANES,), jnp.float32).at[:On].set(b)
    out = _make_pool(Bn, Ln)(data.T, length.astype(jnp.int32), bias, packed)
    return out[:, :On]


# speedup vs baseline: 1.0912x; 1.0912x over previous
import functools

import jax
import jax.numpy as jnp
from jax import lax
from jax.experimental import pallas as pl
from jax.experimental.pallas import tpu as pltpu
from jax.experimental.pallas import tpu_sc as plsc

_NC = 2
_NS = 16
_NW = _NC * _NS
_LANES = 16
_OPAD = 8


def _rne_bf16_bits(x):
    u = lax.bitcast_convert_type(x, jnp.int32)
    lsb = lax.shift_right_logical(u, 16) & 1
    return lax.shift_right_logical(u + 0x7FFF + lsb, 16)


def _make_proj(Vn, Dn, blk):
    grid = (Vn + blk - 1) // blk

    def body(t_ref, w_ref, o_ref):
        i = pl.program_id(0)
        cols = lax.broadcasted_iota(jnp.int32, (1, blk), 1) + i * blk
        t = jnp.where(cols == 0, 0.0, t_ref[...])
        o = lax.dot_general(w_ref[...], t, (((1,), (0,)), ((), ())),
                            preferred_element_type=jnp.float32)
        b0 = _rne_bf16_bits(o[0, :])
        b1 = _rne_bf16_bits(o[1, :])
        o_ref[...] = b0 | lax.shift_left(b1, 16)

    return pl.pallas_call(
        body,
        grid=(grid,),
        in_specs=[
            pl.BlockSpec((Dn, blk), lambda i: (0, i)),
            pl.BlockSpec((_OPAD, Dn), lambda i: (0, 0)),
        ],
        out_specs=pl.BlockSpec((blk,), lambda i: (i,)),
        out_shape=jax.ShapeDtypeStruct((Vn,), jnp.int32),
    )


def _make_pool(Bn, Ln):
    bpw = Bn // _NW
    mesh = plsc.VectorSubcoreMesh(
        core_axis_name="c", subcore_axis_name="s",
        num_cores=_NC, num_subcores=_NS)

    chunks = []
    off = 0
    while off < Ln:
        sz = min(128, Ln - off)
        chunks.append((off, sz))
        off += sz

    nacc = (Ln + _LANES - 1) // _LANES
    lpad = nacc * _LANES
    nbuf = 8
    assert bpw % nbuf == 0

    @functools.partial(
        pl.kernel,
        out_type=jax.ShapeDtypeStruct((Bn, _LANES), jnp.float32),
        mesh=mesh,
        scratch_types=[
            pltpu.VMEM((bpw, Ln), jnp.int32),
            pltpu.VMEM((nbuf, lpad), jnp.int32),
            pltpu.VMEM((bpw,), jnp.int32),
            pltpu.VMEM((_LANES,), jnp.float32),
            pltpu.VMEM((bpw, _LANES), jnp.float32),
            [pltpu.SemaphoreType.DMA] * nbuf,
        ],
        compiler_params=pltpu.CompilerParams(
            use_tc_tiling_on_sc=False, needs_layout_passes=False),
    )
    def pool(data_hbm, len_hbm, bias_hbm, packed_hbm, out_hbm,
             idx_v, rows_v, len_v, bias_v, out_v, sems):
        wid = lax.axis_index("s") * _NC + lax.axis_index("c")
        base = wid * bpw
        pltpu.sync_copy(data_hbm.at[pl.ds(base, bpw)], idx_v)
        pltpu.sync_copy(len_hbm.at[pl.ds(base, bpw)], len_v)
        pltpu.sync_copy(bias_hbm, bias_v)
        bias_vec = bias_v[...]
        lane = lax.iota(jnp.int32, _LANES)

        izeros = jnp.zeros((_LANES,), jnp.int32)
        for slot in range(nbuf):
            for c in range(Ln // _LANES * _LANES, lpad, _LANES):
                rows_v[slot, pl.ds(c, _LANES)] = izeros

        def fire(g, slot, sem):
            gi = jnp.minimum(g, bpw - 1)
            for (o, sz) in chunks:
                pltpu.async_copy(
                    packed_hbm.at[idx_v.at[gi, pl.ds(o, sz)]],
                    rows_v.at[slot, pl.ds(o, sz)], sem)

        def drain(slot, sem):
            for (o, sz) in chunks:
                pltpu.make_async_copy(
                    packed_hbm.at[idx_v.at[0, pl.ds(o, sz)]],
                    rows_v.at[slot, pl.ds(o, sz)], sem).wait()

        fzeros = jnp.zeros((_LANES,), jnp.float32)
        himask = jnp.full((_LANES,), -65536, jnp.int32)

        def acc_store(g, slot):
            acc0 = fzeros
            acc1 = fzeros
            for c in range(nacc):
                w = rows_v[slot, pl.ds(c * _LANES, _LANES)]
                acc0 = acc0 + plsc.bitcast(lax.shift_left(w, 16), jnp.float32)
                acc1 = acc1 + plsc.bitcast(w & himask, jnp.float32)
            s0 = jnp.sum(acc0)
            s1 = jnp.sum(acc1)
            lenf = plsc.load_gather(
                len_v, [jnp.full((_LANES,), g, jnp.int32)]).astype(jnp.float32)
            res = jnp.where(lane == 0, s0, jnp.where(lane == 1, s1, 0.0))
            out_v[g, :] = res / lenf + bias_vec

        for j in range(nbuf - 1):
            fire(j, j, sems[j])

        def outer(i, carry):
            g = nbuf * i
            for j in range(nbuf):
                fire(g + nbuf - 1 + j, (j + nbuf - 1) % nbuf,
                     sems[(j + nbuf - 1) % nbuf])
                drain(j, sems[j])
                acc_store(g + j, j)
            return carry

        lax.fori_loop(0, bpw // nbuf, outer, 0)
        for j in range(nbuf - 1):
            drain(j, sems[j])

        pltpu.sync_copy(out_v, out_hbm.at[pl.ds(base, bpw)])

    return pool


def kernel(data, length, embed_table, W, b):
    Bn, Ln = data.shape
    Vn, Dn = embed_table.shape
    On = W.shape[0]
    data = data.astype(jnp.int32)

    Wp = jnp.zeros((_OPAD, Dn), jnp.float32).at[:On].set(W)
    packed = _make_proj(Vn, Dn, 65536)(embed_table.T, Wp)

    bias = jnp.zeros((_LANES,), jnp.float32).at[:On].set(b)
    out = _make_pool(Bn, Ln)(data, length.astype(jnp.int32), bias, packed)
    return out[:, :On]
